# baseline (speedup 1.0000x reference)
import functools

import jax
import jax.numpy as jnp
from jax import lax
from jax.experimental import pallas as pl
from jax.experimental.pallas import tpu as pltpu
from jax.experimental.pallas import tpu_sc as plsc

OUT_VOCAB = 16
OUT_LEN = 8
D = OUT_VOCAB * OUT_LEN


def _gather_body(x_hbm, emb_hbm, out_hbm, idx_v, row_v, sem):
    wid = lax.axis_index("s") * 2 + lax.axis_index("c")

    @pl.when(wid == 0)
    def _():
        pltpu.sync_copy(x_hbm, idx_v)
        pltpu.async_copy(emb_hbm.at[idx_v], row_v, sem).wait()
        pltpu.sync_copy(row_v, out_hbm)


@jax.jit
def _sc_gather(x, emb):
    mesh = plsc.VectorSubcoreMesh(core_axis_name="c", subcore_axis_name="s")
    run = functools.partial(
        pl.kernel,
        mesh=mesh,
        out_type=jax.ShapeDtypeStruct((1, D), jnp.float32),
        scratch_types=[
            pltpu.VMEM((1,), jnp.int32),
            pltpu.VMEM((1, D), jnp.float32),
            pltpu.SemaphoreType.DMA,
        ],
    )(_gather_body)
    return run(x, emb)


def kernel(x, emb, lproj_w, rproj_w):
    row = _sc_gather(x, emb)
    return row.reshape(OUT_VOCAB, OUT_LEN)


# speedup vs baseline: 1.1056x; 1.1056x over previous
import functools

import jax
import jax.numpy as jnp
from jax import lax
from jax.experimental import pallas as pl
from jax.experimental.pallas import tpu as pltpu
from jax.experimental.pallas import tpu_sc as plsc

OUT_VOCAB = 16
OUT_LEN = 8
D = OUT_VOCAB * OUT_LEN


def _gather_body(x_hbm, emb_hbm, out_hbm, idx_v, row_v, sem):
    wid = lax.axis_index("s") * 2 + lax.axis_index("c")

    @pl.when(wid == 0)
    def _():
        pltpu.sync_copy(x_hbm, idx_v)
        pltpu.async_copy(emb_hbm.at[idx_v], row_v, sem).wait()
        pltpu.sync_copy(row_v, out_hbm)


@jax.jit
def _sc_gather(x, emb):
    mesh = plsc.VectorSubcoreMesh(
        core_axis_name="c", subcore_axis_name="s", num_cores=1
    )
    run = functools.partial(
        pl.kernel,
        mesh=mesh,
        out_type=jax.ShapeDtypeStruct((1, D), jnp.float32),
        scratch_types=[
            pltpu.VMEM((1,), jnp.int32),
            pltpu.VMEM((1, D), jnp.float32),
            pltpu.SemaphoreType.DMA,
        ],
    )(_gather_body)
    return run(x, emb)


def kernel(x, emb, lproj_w, rproj_w):
    row = _sc_gather(x, emb)
    return row.reshape(OUT_VOCAB, OUT_LEN)


# speedup vs baseline: 1.1696x; 1.0579x over previous
import functools

import jax
import jax.numpy as jnp
from jax.experimental import pallas as pl
from jax.experimental.pallas import tpu as pltpu
from jax.experimental.pallas import tpu_sc as plsc

OUT_VOCAB = 16
OUT_LEN = 8
D = OUT_VOCAB * OUT_LEN


def _gather_body(x_hbm, emb_hbm, out_hbm, idx_s):
    pltpu.sync_copy(x_hbm, idx_s)
    i = idx_s[0]
    pltpu.sync_copy(emb_hbm.at[pl.ds(i, 1)], out_hbm)


@jax.jit
def _sc_gather(x, emb):
    mesh = plsc.ScalarSubcoreMesh(axis_name="c", num_cores=1)
    run = functools.partial(
        pl.kernel,
        mesh=mesh,
        out_type=jax.ShapeDtypeStruct((1, D), jnp.float32),
        scratch_types=[
            pltpu.SMEM((1,), jnp.int32),
        ],
    )(_gather_body)
    return run(x, emb)


def kernel(x, emb, lproj_w, rproj_w):
    row = _sc_gather(x, emb)
    return row.reshape(OUT_VOCAB, OUT_LEN)
